# baseline (speedup 1.0000x reference)
import functools

import jax
import jax.numpy as jnp
from jax import lax
from jax.experimental import pallas as pl
from jax.experimental.pallas import tpu as pltpu
from jax.experimental.pallas import tpu_sc as plsc

CHAR_DIM = 32
WORD_DIM = 32
OUT_DIM = CHAR_DIM + WORD_DIM
IPR = 128
NB = 2
NSLOTS = 3


@functools.lru_cache(maxsize=None)
def _make_sc_kernel(n_batch: int, seq_len: int):
    info = plsc.get_sparse_core_info()
    nw = info.num_cores * info.num_subcores
    assert n_batch % (nw * NB) == 0
    batches_per_w = n_batch // nw
    n_iter = batches_per_w // NB
    assert n_iter >= NSLOTS
    nc = info.num_cores
    n_tokens = n_batch * seq_len
    chunk_tok = NB * seq_len
    splits = []
    off = 0
    while off < seq_len:
        splits.append((off, min(IPR, seq_len - off)))
        off += min(IPR, seq_len - off)

    mesh = plsc.VectorSubcoreMesh(core_axis_name="c", subcore_axis_name="s")

    @functools.partial(
        pl.kernel,
        mesh=mesh,
        out_type=jax.ShapeDtypeStruct((n_tokens, OUT_DIM), jnp.float32),
        compiler_params=pltpu.CompilerParams(use_tc_tiling_on_sc=False),
        scratch_types=[
            pltpu.VMEM((NSLOTS, NB * 2 * seq_len), jnp.int32),
            pltpu.VMEM((NSLOTS, chunk_tok, CHAR_DIM), jnp.float32),
            pltpu.VMEM((NSLOTS, chunk_tok, WORD_DIM), jnp.float32),
            pltpu.SemaphoreType.DMA((NSLOTS,)),
            pltpu.SemaphoreType.DMA((NSLOTS,)),
        ],
    )
    def k(voc_hbm, char_hbm, word_hbm, out_hbm, iv_v, cb_v, wb_v,
          sem_g, sem_w):
        wid = lax.axis_index("s") * nc + lax.axis_index("c")
        batch_base = wid * batches_per_w
        tok_base = batch_base * seq_len

        def gather_copies(slot):
            copies = []
            for b in range(NB):
                for (o, n) in splits:
                    copies.append(pltpu.make_async_copy(
                        char_hbm.at[iv_v.at[slot, pl.ds(b * 2 * seq_len + o, n)]],
                        cb_v.at[slot, pl.ds(b * seq_len + o, n)],
                        sem_g.at[slot]))
                    copies.append(pltpu.make_async_copy(
                        word_hbm.at[iv_v.at[slot, pl.ds(b * 2 * seq_len + seq_len + o, n)]],
                        wb_v.at[slot, pl.ds(b * seq_len + o, n)],
                        sem_g.at[slot]))
            return copies

        def issue_gathers(chunk_i, slot):
            w0 = (batch_base + chunk_i * NB) * 2 * seq_len
            pltpu.sync_copy(voc_hbm.at[pl.ds(w0, NB * 2 * seq_len)],
                            iv_v.at[slot])
            for c in gather_copies(slot):
                c.start()

        def write_copies(chunk_i, slot):
            tok0 = tok_base + chunk_i * chunk_tok
            return [
                pltpu.make_async_copy(
                    cb_v.at[slot],
                    out_hbm.at[pl.ds(tok0, chunk_tok), pl.ds(0, CHAR_DIM)],
                    sem_w.at[slot]),
                pltpu.make_async_copy(
                    wb_v.at[slot],
                    out_hbm.at[pl.ds(tok0, chunk_tok), pl.ds(CHAR_DIM, WORD_DIM)],
                    sem_w.at[slot]),
            ]

        for p in range(NSLOTS - 1):
            issue_gathers(p, p)

        def body(i, carry):
            slot = lax.rem(i, NSLOTS)
            for c in gather_copies(slot):
                c.wait()
            for c in write_copies(i, slot):
                c.start()

            @pl.when(i + NSLOTS - 1 < n_iter)
            def _issue_ahead():
                nslot = lax.rem(i + NSLOTS - 1, NSLOTS)

                @pl.when(i > 0)
                def _drain_stale_write():
                    for c in write_copies(i - 1, nslot):
                        c.wait()

                issue_gathers(i + NSLOTS - 1, nslot)

            return carry

        lax.fori_loop(0, n_iter, body, 0)
        for tail in range(NSLOTS, 0, -1):
            for c in write_copies(n_iter - tail, (n_iter - tail) % NSLOTS):
                c.wait()

    return k


def kernel(voc, char_table, word_table):
    b, _, l = voc.shape
    if voc.dtype != jnp.int32:
        voc = voc.astype(jnp.int32)
    voc_flat = voc.reshape(-1)
    out = _make_sc_kernel(b, l)(voc_flat, char_table, word_table)
    return out.reshape(b, l, OUT_DIM)


# speedup vs baseline: 1.0011x; 1.0011x over previous
import functools

import jax
import jax.numpy as jnp
from jax import lax
from jax.experimental import pallas as pl
from jax.experimental.pallas import tpu as pltpu
from jax.experimental.pallas import tpu_sc as plsc

CHAR_DIM = 32
WORD_DIM = 32
OUT_DIM = CHAR_DIM + WORD_DIM
IPR = 128
NB = 2
NSLOTS = 3


@functools.lru_cache(maxsize=None)
def _make_sc_kernel(n_batch: int, seq_len: int):
    info = plsc.get_sparse_core_info()
    nw = info.num_cores * info.num_subcores
    assert n_batch % (nw * NB) == 0
    batches_per_w = n_batch // nw
    n_iter = batches_per_w // NB
    assert n_iter >= NSLOTS
    nc = info.num_cores
    chunk_tok = NB * seq_len
    splits = []
    off = 0
    while off < seq_len:
        splits.append((off, min(IPR, seq_len - off)))
        off += min(IPR, seq_len - off)

    mesh = plsc.VectorSubcoreMesh(core_axis_name="c", subcore_axis_name="s")

    @functools.partial(
        pl.kernel,
        mesh=mesh,
        out_type=jax.ShapeDtypeStruct((n_batch, seq_len, OUT_DIM), jnp.float32),
        compiler_params=pltpu.CompilerParams(use_tc_tiling_on_sc=False),
        scratch_types=[
            pltpu.VMEM((NSLOTS, NB, 2, seq_len), jnp.int32),
            pltpu.VMEM((NSLOTS, NB, seq_len, CHAR_DIM), jnp.float32),
            pltpu.VMEM((NSLOTS, NB, seq_len, WORD_DIM), jnp.float32),
            pltpu.SemaphoreType.DMA((NSLOTS,)),
            pltpu.SemaphoreType.DMA((NSLOTS,)),
        ],
    )
    def k(voc_hbm, char_hbm, word_hbm, out_hbm, iv_v, cb_v, wb_v,
          sem_g, sem_w):
        wid = lax.axis_index("s") * nc + lax.axis_index("c")
        batch_base = wid * batches_per_w

        def gather_copies(slot):
            copies = []
            for b in range(NB):
                for (o, n) in splits:
                    copies.append(pltpu.make_async_copy(
                        char_hbm.at[iv_v.at[slot, b, 0, pl.ds(o, n)]],
                        cb_v.at[slot, b, pl.ds(o, n)],
                        sem_g.at[slot]))
                    copies.append(pltpu.make_async_copy(
                        word_hbm.at[iv_v.at[slot, b, 1, pl.ds(o, n)]],
                        wb_v.at[slot, b, pl.ds(o, n)],
                        sem_g.at[slot]))
            return copies

        def issue_gathers(chunk_i, slot):
            b0 = batch_base + chunk_i * NB
            pltpu.sync_copy(voc_hbm.at[pl.ds(b0, NB)], iv_v.at[slot])
            for c in gather_copies(slot):
                c.start()

        def write_copies(chunk_i, slot):
            b0 = batch_base + chunk_i * NB
            return [
                pltpu.make_async_copy(
                    cb_v.at[slot],
                    out_hbm.at[pl.ds(b0, NB), :, pl.ds(0, CHAR_DIM)],
                    sem_w.at[slot]),
                pltpu.make_async_copy(
                    wb_v.at[slot],
                    out_hbm.at[pl.ds(b0, NB), :, pl.ds(CHAR_DIM, WORD_DIM)],
                    sem_w.at[slot]),
            ]

        for p in range(NSLOTS - 1):
            issue_gathers(p, p)

        def body(i, carry):
            slot = lax.rem(i, NSLOTS)
            for c in gather_copies(slot):
                c.wait()
            for c in write_copies(i, slot):
                c.start()

            @pl.when(i + NSLOTS - 1 < n_iter)
            def _issue_ahead():
                nslot = lax.rem(i + NSLOTS - 1, NSLOTS)

                @pl.when(i > 0)
                def _drain_stale_write():
                    for c in write_copies(i - 1, nslot):
                        c.wait()

                issue_gathers(i + NSLOTS - 1, nslot)

            return carry

        lax.fori_loop(0, n_iter, body, 0)
        for tail in range(NSLOTS, 0, -1):
            for c in write_copies(n_iter - tail, (n_iter - tail) % NSLOTS):
                c.wait()

    return k


def kernel(voc, char_table, word_table):
    b, _, l = voc.shape
    if voc.dtype != jnp.int32:
        voc = voc.astype(jnp.int32)
    return _make_sc_kernel(b, l)(voc, char_table, word_table)
